# Initial kernel scaffold; baseline (speedup 1.0000x reference)
import functools

import jax
import jax.numpy as jnp
from jax import lax
from jax.experimental import pallas as pl
from jax.experimental.pallas import tpu as pltpu
from jax.experimental.pallas import tpu_sc as plsc


def kernel(x, embedding_matrix):
    B, H = x.shape
    V, D = embedding_matrix.shape
    total = B * H

    NW = 32
    K = 8
    CHUNK = K * 128
    per_w = total // NW
    n_chunks = per_w // CHUNK

    x_idx = x.reshape(NW, n_chunks, K, 128).astype(jnp.int32)

    mesh = plsc.VectorSubcoreMesh(core_axis_name="c", subcore_axis_name="s")

    @functools.partial(
        pl.kernel,
        mesh=mesh,
        out_type=jax.ShapeDtypeStruct((total, D), jnp.float32),
        scratch_types=[
            pltpu.VMEM((K, 128), jnp.int32),
            pltpu.VMEM((CHUNK, D), jnp.float32),
            pltpu.SemaphoreType.DMA,
        ],
    )
    def sc_gather(table_hbm, idx_hbm, out_hbm, idx_v, rows_v, gsem):
        wid = lax.axis_index("s") * 2 + lax.axis_index("c")

        def body(i, carry):
            pltpu.sync_copy(idx_hbm.at[wid, i], idx_v)
            copies = [
                pltpu.async_copy(
                    table_hbm.at[idx_v.at[j]],
                    rows_v.at[pl.ds(j * 128, 128)],
                    gsem,
                )
                for j in range(K)
            ]
            for c in copies:
                c.wait()
            off = wid * per_w + i * CHUNK
            pltpu.sync_copy(rows_v, out_hbm.at[pl.ds(off, CHUNK)])
            return carry

        lax.fori_loop(0, n_chunks, body, 0)

    out = sc_gather(embedding_matrix, x_idx)
    return out.reshape(B, H, D)


# speedup vs baseline: 4.8098x; 4.8098x over previous
import functools

import jax
import jax.numpy as jnp
from jax import lax
from jax.experimental import pallas as pl
from jax.experimental.pallas import tpu as pltpu
from jax.experimental.pallas import tpu_sc as plsc


def kernel(x, embedding_matrix):
    B, H = x.shape
    V, D = embedding_matrix.shape
    total = B * H

    NW = 32
    K = 8
    CHUNK = K * 128
    per_w = total // NW
    n_chunks = per_w // CHUNK

    x_idx = x.reshape(NW, n_chunks, K, 128).astype(jnp.int32)

    mesh = plsc.VectorSubcoreMesh(core_axis_name="c", subcore_axis_name="s")

    @functools.partial(
        pl.kernel,
        mesh=mesh,
        out_type=jax.ShapeDtypeStruct((total, D), jnp.float32),
        scratch_types=[
            pltpu.VMEM((K, 128), jnp.int32),
            pltpu.VMEM((CHUNK, D), jnp.float32),
            pltpu.SemaphoreType.DMA,
        ],
        compiler_params=pltpu.CompilerParams(use_tc_tiling_on_sc=False),
    )
    def sc_gather(table_hbm, idx_hbm, out_hbm, idx_v, rows_v, gsem):
        wid = lax.axis_index("s") * 2 + lax.axis_index("c")

        def body(i, carry):
            pltpu.sync_copy(idx_hbm.at[wid, i], idx_v)
            copies = [
                pltpu.async_copy(
                    table_hbm.at[idx_v.at[j]],
                    rows_v.at[pl.ds(j * 128, 128)],
                    gsem,
                )
                for j in range(K)
            ]
            for c in copies:
                c.wait()
            off = wid * per_w + i * CHUNK
            pltpu.sync_copy(rows_v, out_hbm.at[pl.ds(off, CHUNK)])
            return carry

        lax.fori_loop(0, n_chunks, body, 0)

    out = sc_gather(embedding_matrix, x_idx)
    return out.reshape(B, H, D)


# speedup vs baseline: 4.9685x; 1.0330x over previous
import functools

import jax
import jax.numpy as jnp
from jax import lax
from jax.experimental import pallas as pl
from jax.experimental.pallas import tpu as pltpu
from jax.experimental.pallas import tpu_sc as plsc


def kernel(x, embedding_matrix):
    B, H = x.shape
    V, D = embedding_matrix.shape
    total = B * H

    NW = 32
    K = 10
    CHUNK = K * 128
    NSLOT = 2
    per_w = total // NW
    n_chunks = per_w // CHUNK
    n_rot = n_chunks // NSLOT

    x_idx = x.reshape(NW, n_chunks, K, 128).astype(jnp.int32)

    mesh = plsc.VectorSubcoreMesh(core_axis_name="c", subcore_axis_name="s")

    @functools.partial(
        pl.kernel,
        mesh=mesh,
        out_type=jax.ShapeDtypeStruct((total, D), jnp.float32),
        scratch_types=[
            pltpu.VMEM((NSLOT, K, 128), jnp.int32),
            pltpu.VMEM((NSLOT, CHUNK, D), jnp.float32),
        ] + [pltpu.SemaphoreType.DMA] * (2 * NSLOT),
        compiler_params=pltpu.CompilerParams(use_tc_tiling_on_sc=False),
    )
    def sc_gather(table_hbm, idx_hbm, out_hbm, idx_v, rows_v, *sems):
        gsem = sems[:NSLOT]
        osem = sems[NSLOT:]
        wid = lax.axis_index("s") * 2 + lax.axis_index("c")
        base = wid * per_w

        def fire(b, i):
            pltpu.sync_copy(idx_hbm.at[wid, i], idx_v.at[b])
            for j in range(K):
                pltpu.async_copy(
                    table_hbm.at[idx_v.at[b, j]],
                    rows_v.at[b, pl.ds(j * 128, 128)],
                    gsem[b],
                )

        def wait_gathers(b):
            pltpu.make_async_copy(
                out_hbm.at[pl.ds(0, CHUNK)], rows_v.at[b], gsem[b]
            ).wait()

        for b in range(NSLOT):
            fire(b, b)

        def body(p, carry):
            for b in range(NSLOT):
                i = p * NSLOT + b
                wait_gathers(b)
                pltpu.async_copy(
                    rows_v.at[b], out_hbm.at[pl.ds(base + i * CHUNK, CHUNK)],
                    osem[b],
                ).wait()
                fire(b, i + NSLOT)
            return carry

        lax.fori_loop(0, n_rot - 1, body, 0)

        for b in range(NSLOT):
            i = (n_rot - 1) * NSLOT + b
            wait_gathers(b)
            pltpu.async_copy(
                rows_v.at[b], out_hbm.at[pl.ds(base + i * CHUNK, CHUNK)],
                osem[b],
            ).wait()

    out = sc_gather(embedding_matrix, x_idx)
    return out.reshape(B, H, D)
